# Initial kernel scaffold; baseline (speedup 1.0000x reference)
import jax
import jax.numpy as jnp
from jax.experimental import pallas as pl


def kernel(input_ids, table, proj_w):
    raise NotImplementedError("write your pallas kernel here")


# speedup vs baseline: 1.0142x; 1.0142x over previous
import functools

import jax
import jax.numpy as jnp
from jax import lax
from jax.experimental import pallas as pl
from jax.experimental.pallas import tpu as pltpu
from jax.experimental.pallas import tpu_sc as plsc

_NUM_BUCKETS = 1000000
_HASH_DIM = 64
_MODEL_DIM = 1024
_MULT = 92821
_MULT_HI = (_MULT * 1024) % _NUM_BUCKETS

_NC = 2
_NS = 16
_NW = _NC * _NS
_LANES = 16
_CHUNK = 64


def _sc_hash_gather(tokens_per_worker):
    n_chunks = tokens_per_worker // _CHUNK

    def body(ids_hbm, prev_hbm, table_hbm, emb_hbm, ids_v, prev_v, hbuf_v,
             tiles_v, emb_v, sem):
        wid = lax.axis_index("s") * _NC + lax.axis_index("c")
        base = wid * tokens_per_worker
        pltpu.sync_copy(ids_hbm.at[pl.ds(base, tokens_per_worker)], ids_v)
        pltpu.sync_copy(prev_hbm.at[pl.ds(base, tokens_per_worker)], prev_v)
        nbv = jnp.full((_LANES,), _NUM_BUCKETS, jnp.int32)
        for k in range(tokens_per_worker // _LANES):
            off = k * _LANES
            ids16 = ids_v[pl.ds(off, _LANES)]
            prev16 = prev_v[pl.ds(off, _LANES)]
            p = lax.rem(prev16, nbv)
            p_hi = lax.shift_right_logical(
                p, jnp.full((_LANES,), 10, jnp.int32))
            p_lo = lax.bitwise_and(p, jnp.full((_LANES,), 1023, jnp.int32))
            term = lax.rem(p_hi * _MULT_HI, nbv)
            term = lax.rem(term + lax.rem(p_lo * _MULT, nbv), nbv)
            h = lax.rem(term + lax.rem(ids16, nbv), nbv)
            hbuf_v[pl.ds(off, _LANES)] = h

        lanes = lax.iota(jnp.int32, _LANES)
        for c in range(n_chunks):
            copies = []
            for k in range(_CHUNK // _LANES):
                h16 = hbuf_v[pl.ds(c * _CHUNK + k * _LANES, _LANES)]
                tid16 = lax.shift_right_logical(
                    h16, jnp.full((_LANES,), 3, jnp.int32))
                for j in range(_LANES):
                    slot = k * _LANES + j
                    copies.append(pltpu.async_copy(
                        table_hbm.at[tid16[j]], tiles_v.at[slot], sem))
            for cp in copies:
                cp.wait()
            for g in range(_CHUNK // _LANES):
                off = c * _CHUNK + g * _LANES
                h16 = hbuf_v[pl.ds(off, _LANES)]
                row16 = lax.bitwise_and(h16, jnp.full((_LANES,), 7, jnp.int32))
                slot16 = lanes + g * _LANES
                tok16 = lanes + off

                def dstep(d, _):
                    d16 = jnp.zeros((_LANES,), jnp.int32) + d
                    vals = plsc.load_gather(tiles_v, [slot16, row16, d16])
                    plsc.store_scatter(emb_v, [tok16, d16], vals)
                    return 0

                lax.fori_loop(0, _HASH_DIM, dstep, 0)
        pltpu.sync_copy(emb_v, emb_hbm.at[pl.ds(base, tokens_per_worker)])

    return body


def _proj_body(emb_ref, w_ref, out_ref):
    out_ref[...] = lax.dot_general(
        emb_ref[...], w_ref[...],
        (((1,), (1,)), ((), ())),
        preferred_element_type=jnp.float32,
    )


@jax.jit
def kernel(input_ids, table, proj_w):
    batch, seq = input_ids.shape
    tok = batch * seq
    tokens_per_worker = tok // _NW

    ids = input_ids.astype(jnp.int32)
    prev = jnp.concatenate([jnp.zeros_like(ids[:, :1]), ids[:, :-1]], axis=1)
    ids_flat = ids.reshape(tok)
    prev_flat = prev.reshape(tok)
    table3 = table.reshape(_NUM_BUCKETS // 8, 8, _HASH_DIM)

    sc_gather = pl.kernel(
        _sc_hash_gather(tokens_per_worker),
        out_type=jax.ShapeDtypeStruct((tok, _HASH_DIM), jnp.float32),
        mesh=plsc.VectorSubcoreMesh(core_axis_name="c", subcore_axis_name="s"),
        compiler_params=pltpu.CompilerParams(
            use_tc_tiling_on_sc=True, needs_layout_passes=False),
        scratch_types=[
            pltpu.VMEM((tokens_per_worker,), jnp.int32),
            pltpu.VMEM((tokens_per_worker,), jnp.int32),
            pltpu.VMEM((tokens_per_worker,), jnp.int32),
            pltpu.VMEM((_CHUNK, 8, _HASH_DIM), jnp.float32),
            pltpu.VMEM((tokens_per_worker, _HASH_DIM), jnp.float32),
            pltpu.SemaphoreType.DMA,
        ],
    )
    emb = sc_gather(ids_flat, prev_flat, table3)

    blk = 1024
    out = pl.pallas_call(
        _proj_body,
        grid=(tok // blk,),
        in_specs=[
            pl.BlockSpec((blk, _HASH_DIM), lambda i: (i, 0)),
            pl.BlockSpec((_MODEL_DIM, _HASH_DIM), lambda i: (0, 0)),
        ],
        out_specs=pl.BlockSpec((blk, _MODEL_DIM), lambda i: (i, 0)),
        out_shape=jax.ShapeDtypeStruct((tok, _MODEL_DIM), jnp.float32),
    )(emb, proj_w)

    return out.reshape(batch, seq, _MODEL_DIM)
